# Initial kernel scaffold; baseline (speedup 1.0000x reference)
import jax
import jax.numpy as jnp
from jax.experimental import pallas as pl


def kernel(src_tokens, dst_tokens, src_table, dst_table):
    raise NotImplementedError("write your pallas kernel here")


# speedup vs baseline: 3.5243x; 3.5243x over previous
import functools

import jax
import jax.numpy as jnp
from jax import lax
from jax.experimental import pallas as pl
from jax.experimental.pallas import tpu as pltpu
from jax.experimental.pallas import tpu_sc as plsc

SRC_VOCAB = 100000
DST_VOCAB = 100000
EMBED = 128
B = 4096 * 50
NC, NS = 2, 16
NW = NC * NS
B_PER_W = B // NW
CHUNK = 800
N_CHUNKS = B_PER_W // CHUNK


def _emb_kernel(src_idx, dst_idx, src_table, dst_table,
                src_out, dst_out, idx_v, rows_v, sem):
    wid = lax.axis_index("s") * NC + lax.axis_index("c")
    base_w = wid * B_PER_W
    for idx_hbm, table_hbm, out_hbm in (
        (src_idx, src_table, src_out),
        (dst_idx, dst_table, dst_out),
    ):
        for c in range(N_CHUNKS):
            base = base_w + c * CHUNK
            pltpu.sync_copy(idx_hbm.at[pl.ds(base, CHUNK)], idx_v)
            pltpu.async_copy(table_hbm.at[idx_v], rows_v, sem).wait()
            pltpu.sync_copy(rows_v, out_hbm.at[pl.ds(base, CHUNK)])


@jax.jit
def kernel(src_tokens, dst_tokens, src_table, dst_table):
    shape = src_tokens.shape
    src_flat = src_tokens.reshape(B).astype(jnp.int32)
    dst_flat = dst_tokens.reshape(B).astype(jnp.int32)

    mesh = plsc.VectorSubcoreMesh(core_axis_name="c", subcore_axis_name="s")
    run = pl.kernel(
        _emb_kernel,
        out_type=(
            jax.ShapeDtypeStruct((B, EMBED), jnp.float32),
            jax.ShapeDtypeStruct((B, EMBED), jnp.float32),
        ),
        mesh=mesh,
        scratch_types=[
            pltpu.VMEM((CHUNK,), jnp.int32),
            pltpu.VMEM((CHUNK, EMBED), jnp.float32),
            pltpu.SemaphoreType.DMA,
        ],
    )
    src_emb, dst_emb = run(src_flat, dst_flat, src_table, dst_table)
    return (src_emb.reshape(*shape, EMBED), dst_emb.reshape(*shape, EMBED))


# speedup vs baseline: 3.5648x; 1.0115x over previous
import jax
import jax.numpy as jnp
from jax import lax
from jax.experimental import pallas as pl
from jax.experimental.pallas import tpu as pltpu
from jax.experimental.pallas import tpu_sc as plsc

EMBED = 128
B = 4096 * 50
NC, NS = 2, 16
NW = NC * NS
B_PER_W = B // NW
CHUNK = 400
N_CHUNKS = B_PER_W // CHUNK
NB = 2


def _emb_kernel(src_idx, dst_idx, src_table, dst_table,
                src_out, dst_out, idx_s, idx_d, rows, gsem, wsem):
    wid = lax.axis_index("s") * NC + lax.axis_index("c")
    base_w = wid * B_PER_W
    pltpu.sync_copy(src_idx.at[pl.ds(base_w, B_PER_W)], idx_s)
    pltpu.sync_copy(dst_idx.at[pl.ds(base_w, B_PER_W)], idx_d)

    for idx_v, table, out in ((idx_s, src_table, src_out),
                              (idx_d, dst_table, dst_out)):
        def g_start(c, b, table=table, idx_v=idx_v):
            pltpu.async_copy(
                table.at[idx_v.at[pl.ds(c * CHUNK, CHUNK)]],
                rows.at[b], gsem.at[b])

        def g_wait(b, table=table, idx_v=idx_v):
            pltpu.make_async_copy(
                table.at[idx_v.at[pl.ds(0, CHUNK)]],
                rows.at[b], gsem.at[b]).wait()

        def w_start(c, b, out=out):
            pltpu.async_copy(
                rows.at[b], out.at[pl.ds(base_w + c * CHUNK, CHUNK)],
                wsem.at[b])

        def w_wait(b, out=out):
            pltpu.make_async_copy(
                rows.at[b], out.at[pl.ds(base_w, CHUNK)],
                wsem.at[b]).wait()

        for b in range(NB):
            g_start(b, b)

        @pl.loop(NB, N_CHUNKS, step=NB)
        def _(g):
            for b in range(NB):
                g_wait(b)
                w_start(g - NB + b, b)
                w_wait(b)
                g_start(g + b, b)

        for b in range(NB):
            g_wait(b)
            w_start(N_CHUNKS - NB + b, b)
            w_wait(b)


@jax.jit
def kernel(src_tokens, dst_tokens, src_table, dst_table):
    shape = src_tokens.shape
    src_flat = src_tokens.reshape(B).astype(jnp.int32)
    dst_flat = dst_tokens.reshape(B).astype(jnp.int32)

    mesh = plsc.VectorSubcoreMesh(core_axis_name="c", subcore_axis_name="s")
    run = pl.kernel(
        _emb_kernel,
        out_type=(
            jax.ShapeDtypeStruct((B, EMBED), jnp.float32),
            jax.ShapeDtypeStruct((B, EMBED), jnp.float32),
        ),
        mesh=mesh,
        scratch_types=[
            pltpu.VMEM((B_PER_W,), jnp.int32),
            pltpu.VMEM((B_PER_W,), jnp.int32),
            pltpu.VMEM((NB, CHUNK, EMBED), jnp.float32),
            pltpu.SemaphoreType.DMA((NB,)),
            pltpu.SemaphoreType.DMA((NB,)),
        ],
    )
    src_emb, dst_emb = run(src_flat, dst_flat, src_table, dst_table)
    return (src_emb.reshape(*shape, EMBED), dst_emb.reshape(*shape, EMBED))


# speedup vs baseline: 5.9605x; 1.6720x over previous
import jax
import jax.numpy as jnp
from jax import lax
from jax.experimental import pallas as pl
from jax.experimental.pallas import tpu as pltpu
from jax.experimental.pallas import tpu_sc as plsc

EMBED = 128
NSENT = 4096
SLEN = 50
SPAD = 56
NC, NS = 2, 16
NW = NC * NS
S_PER_W = NSENT // NW
S_CHUNK = 8
N_CHUNKS = S_PER_W // S_CHUNK
NB = 2


def _emb_kernel(src_idx, dst_idx, src_table, dst_table,
                src_out, dst_out, idx_v, rows, gsem, wsem):
    wid = lax.axis_index("s") * NC + lax.axis_index("c")
    sent0 = wid * S_PER_W

    for idx_hbm, table, out in ((src_idx, src_table, src_out),
                                (dst_idx, dst_table, dst_out)):
        pltpu.sync_copy(idx_hbm.at[pl.ds(sent0 * SPAD, S_PER_W * SPAD)],
                        idx_v)

        def g_start(c, b, table=table):
            for k in range(S_CHUNK):
                pltpu.async_copy(
                    table.at[idx_v.at[pl.ds((c * S_CHUNK + k) * SPAD, SLEN)]],
                    rows.at[b, k], gsem.at[b])

        def g_wait(b, table=table):
            for k in range(S_CHUNK):
                pltpu.make_async_copy(
                    table.at[idx_v.at[pl.ds(0, SLEN)]],
                    rows.at[b, k], gsem.at[b]).wait()

        def w_start(c, b, out=out):
            pltpu.async_copy(
                rows.at[b], out.at[pl.ds(sent0 + c * S_CHUNK, S_CHUNK)],
                wsem.at[b])

        def w_wait(b, out=out):
            pltpu.make_async_copy(
                rows.at[b], out.at[pl.ds(sent0, S_CHUNK)],
                wsem.at[b]).wait()

        for b in range(NB):
            g_start(b, b)

        @pl.loop(NB, N_CHUNKS, step=NB)
        def _(g):
            for b in range(NB):
                g_wait(b)
                w_start(g - NB + b, b)
                w_wait(b)
                g_start(g + b, b)

        for b in range(NB):
            g_wait(b)
            w_start(N_CHUNKS - NB + b, b)
            w_wait(b)


@jax.jit
def kernel(src_tokens, dst_tokens, src_table, dst_table):
    src_pad = jnp.pad(src_tokens.astype(jnp.int32), ((0, 0), (0, SPAD - SLEN)))
    dst_pad = jnp.pad(dst_tokens.astype(jnp.int32), ((0, 0), (0, SPAD - SLEN)))

    mesh = plsc.VectorSubcoreMesh(core_axis_name="c", subcore_axis_name="s")
    run = pl.kernel(
        _emb_kernel,
        out_type=(
            jax.ShapeDtypeStruct((NSENT, SLEN, EMBED), jnp.float32),
            jax.ShapeDtypeStruct((NSENT, SLEN, EMBED), jnp.float32),
        ),
        mesh=mesh,
        scratch_types=[
            pltpu.VMEM((S_PER_W * SPAD,), jnp.int32),
            pltpu.VMEM((NB, S_CHUNK, SLEN, EMBED), jnp.float32),
            pltpu.SemaphoreType.DMA((NB,)),
            pltpu.SemaphoreType.DMA((NB,)),
        ],
    )
    return run(src_pad.reshape(NSENT * SPAD), dst_pad.reshape(NSENT * SPAD),
               src_table, dst_table)
